# baseline (speedup 1.0000x reference)
import functools

import jax
import jax.numpy as jnp
from jax import lax
from jax.experimental import pallas as pl
from jax.experimental.pallas import tpu as pltpu
from jax.experimental.pallas import tpu_sc as plsc

_L = 16


def _sum_along_sc(x0, x1, i0f, i1f, *, num_workers, chunk):
    B = i0f.shape[0]
    D = x0.shape[1]
    b_per_w = B // num_workers
    n_chunks = b_per_w // chunk
    vecs_per_row = D // _L

    mesh = plsc.VectorSubcoreMesh(core_axis_name="c", subcore_axis_name="s")

    @functools.partial(
        pl.kernel,
        mesh=mesh,
        out_type=jax.ShapeDtypeStruct((B, D), jnp.float32),
        compiler_params=pltpu.CompilerParams(use_tc_tiling_on_sc=False),
        scratch_types=[
            pltpu.VMEM((chunk,), jnp.int32),
            pltpu.VMEM((chunk,), jnp.int32),
            pltpu.VMEM((chunk, D), jnp.float32),
            pltpu.VMEM((chunk, D), jnp.float32),
            pltpu.SemaphoreType.DMA,
            pltpu.SemaphoreType.DMA,
        ],
    )
    def k(x0_hbm, x1_hbm, i0_hbm, i1_hbm, out_hbm, idx0_v, idx1_v, r0_v, r1_v,
          sem0, sem1):
        wid = lax.axis_index("s") * 2 + lax.axis_index("c")
        base = wid * b_per_w

        def chunk_body(c, _):
            off = base + c * chunk
            pltpu.sync_copy(i0_hbm.at[pl.ds(off, chunk)], idx0_v)
            pltpu.sync_copy(i1_hbm.at[pl.ds(off, chunk)], idx1_v)
            cp0 = pltpu.async_copy(x0_hbm.at[idx0_v], r0_v, sem0)
            cp1 = pltpu.async_copy(x1_hbm.at[idx1_v], r1_v, sem1)
            cp0.wait()
            cp1.wait()

            def add_body(r, _):
                for v in range(vecs_per_row):
                    sl = pl.ds(v * _L, _L)
                    r0_v[r, sl] = r0_v[r, sl] + r1_v[r, sl]
                return ()

            lax.fori_loop(0, chunk, add_body, (), unroll=2)
            pltpu.sync_copy(r0_v, out_hbm.at[pl.ds(off, chunk)])
            return ()

        lax.fori_loop(0, n_chunks, chunk_body, ())

    return k(x0, x1, i0f, i1f)


def kernel(x0, x1, i0, i1):
    i0f = i0.reshape((-1,)).astype(jnp.int32)
    i1f = i1.reshape((-1,)).astype(jnp.int32)
    return _sum_along_sc(x0, x1, i0f, i1f, num_workers=32, chunk=128)


# speedup vs baseline: 1.1525x; 1.1525x over previous
import functools

import jax
import jax.numpy as jnp
from jax import lax
from jax.experimental import pallas as pl
from jax.experimental.pallas import tpu as pltpu
from jax.experimental.pallas import tpu_sc as plsc

_L = 16


def _sum_along_sc(x0, x1, i0f, i1f, *, num_workers, chunk, nbuf):
    B = i0f.shape[0] * i0f.shape[1]
    D = x0.shape[1]
    b_per_w = B // num_workers
    n_chunks = b_per_w // chunk
    ng = n_chunks // nbuf
    vecs_per_row = D // _L

    mesh = plsc.VectorSubcoreMesh(core_axis_name="c", subcore_axis_name="s")

    scratch = [
        pltpu.VMEM((n_chunks, chunk), jnp.int32),
        pltpu.VMEM((n_chunks, chunk), jnp.int32),
    ]
    scratch += [pltpu.VMEM((chunk, D), jnp.float32)] * (3 * nbuf)
    scratch += [pltpu.SemaphoreType.DMA] * (2 * nbuf)

    @functools.partial(
        pl.kernel,
        mesh=mesh,
        out_type=jax.ShapeDtypeStruct((B, D), jnp.float32),
        compiler_params=pltpu.CompilerParams(use_tc_tiling_on_sc=False),
        scratch_types=scratch,
    )
    def k(x0_hbm, x1_hbm, i0_hbm, i1_hbm, out_hbm, *s):
        idx0_v, idx1_v = s[0], s[1]
        g0 = s[2:2 + nbuf]
        g1 = s[2 + nbuf:2 + 2 * nbuf]
        ob = s[2 + 2 * nbuf:2 + 3 * nbuf]
        gsem = s[2 + 3 * nbuf:2 + 4 * nbuf]
        osem = s[2 + 4 * nbuf:2 + 5 * nbuf]

        wid = lax.axis_index("s") * 2 + lax.axis_index("c")
        wrow = wid * n_chunks

        pltpu.sync_copy(i0_hbm.at[pl.ds(wrow, n_chunks)], idx0_v)
        pltpu.sync_copy(i1_hbm.at[pl.ds(wrow, n_chunks)], idx1_v)

        def issue(cc, b):
            pltpu.async_copy(x0_hbm.at[idx0_v.at[cc]], g0[b], gsem[b])
            pltpu.async_copy(x1_hbm.at[idx1_v.at[cc]], g1[b], gsem[b])

        def process(cc, b, wait_out):
            pltpu.make_async_copy(x0_hbm.at[idx0_v.at[cc]], g0[b],
                                  gsem[b]).wait()
            pltpu.make_async_copy(x1_hbm.at[idx1_v.at[cc]], g1[b],
                                  gsem[b]).wait()
            if wait_out:
                pltpu.make_async_copy(
                    ob[b], out_hbm.at[pl.ds(0, chunk)], osem[b]).wait()

            def add_body(r, _):
                for v in range(vecs_per_row):
                    sl = pl.ds(v * _L, _L)
                    ob[b][r, sl] = g0[b][r, sl] + g1[b][r, sl]
                return ()

            lax.fori_loop(0, chunk, add_body, (), unroll=4)
            pltpu.async_copy(
                ob[b], out_hbm.at[pl.ds((wrow + cc) * chunk, chunk)], osem[b])

        for b in range(nbuf):
            issue(b, b)
        for b in range(nbuf):
            process(b, b, wait_out=False)
            issue(b + nbuf, b)

        def turn(g, _):
            for b in range(nbuf):
                cc = g * nbuf + b
                process(cc, b, wait_out=True)
                issue(cc + nbuf, b)
            return ()

        lax.fori_loop(1, ng - 1, turn, ())

        for b in range(nbuf):
            process((ng - 1) * nbuf + b, b, wait_out=True)
        for b in range(nbuf):
            pltpu.make_async_copy(
                ob[b], out_hbm.at[pl.ds(0, chunk)], osem[b]).wait()

    i0r = i0f.reshape((num_workers * n_chunks, chunk))
    i1r = i1f.reshape((num_workers * n_chunks, chunk))
    return k(x0, x1, i0r, i1r)


def kernel(x0, x1, i0, i1):
    i0f = i0.astype(jnp.int32)
    i1f = i1.astype(jnp.int32)
    return _sum_along_sc(x0, x1, i0f, i1f, num_workers=32, chunk=128, nbuf=4)
